# baseline (speedup 1.0000x reference)
import functools

import jax
import jax.numpy as jnp
from jax import lax
from jax.experimental import pallas as pl
from jax.experimental.pallas import tpu as pltpu
from jax.experimental.pallas import tpu_sc as plsc

D = 1024
NH = 16
HD = D // NH
EI = 16
ES = 4
HID = 1024
NOBJ = 8
N = 512
T = NOBJ * N
MBLK = 1024
NM = T // MBLK
SB = 512
NB = T // SB
BLKM = 256
CAP = 2 * T + EI * BLKM
NBLK = CAP // BLKM
NW = 32
CHUNK = 64
NCH = 2 * T // (NW * CHUNK)


def _ln(x, g, b):
    mu = jnp.mean(x, axis=-1, keepdims=True)
    xc = x - mu
    var = jnp.mean(xc * xc, axis=-1, keepdims=True)
    return xc * jax.lax.rsqrt(var + 1e-5) * g + b


def _attn_body(x_ref, lg_ref, lb_ref, wq_ref, bq_ref, wk_ref, bk_ref,
               wv_ref, bv_ref, wo_ref, bo_ref, x1_ref, t_ref, tb_ref):
    x = x_ref[0]
    g = lg_ref[...]
    b = lb_ref[...]
    xn = _ln(x, g, b).astype(jnp.bfloat16)
    q = (jnp.dot(xn, wq_ref[...].astype(jnp.bfloat16),
                 preferred_element_type=jnp.float32)
         + bq_ref[...]).astype(jnp.bfloat16)
    k = (jnp.dot(xn, wk_ref[...].astype(jnp.bfloat16),
                 preferred_element_type=jnp.float32)
         + bk_ref[...]).astype(jnp.bfloat16)
    v = (jnp.dot(xn, wv_ref[...].astype(jnp.bfloat16),
                 preferred_element_type=jnp.float32)
         + bv_ref[...]).astype(jnp.bfloat16)
    scale = 1.0 / (HD ** 0.5)
    ohs = []
    for h in range(NH):
        sl = slice(h * HD, (h + 1) * HD)
        qh = q[:, sl]
        kh = k[:, sl]
        vh = v[:, sl]
        s = lax.dot_general(qh, kh, (((1,), (1,)), ((), ())),
                            preferred_element_type=jnp.float32) * scale
        s = s - jnp.max(s, axis=-1, keepdims=True)
        p = jnp.exp(s)
        p = p / jnp.sum(p, axis=-1, keepdims=True)
        oh = jnp.dot(p.astype(jnp.bfloat16), vh,
                     preferred_element_type=jnp.float32)
        ohs.append(oh.astype(jnp.bfloat16))
    o = jnp.concatenate(ohs, axis=1)
    xa = jnp.dot(o, wo_ref[...].astype(jnp.bfloat16),
                 preferred_element_type=jnp.float32) + bo_ref[...]
    x1 = xa + x
    x1_ref[0] = x1
    t = _ln(x1, g, b)
    t_ref[0] = t
    tb_ref[0] = t.astype(jnp.bfloat16)


def _router_body(t_ref, gate_ref, wcol_ref, psrep_ref, aux_ref, dest_ref,
                 bexp_ref):
    t = t_ref[...]
    logits = jnp.dot(t, gate_ref[...], preferred_element_type=jnp.float32)
    ci = lax.broadcasted_iota(jnp.int32, (T, 128), 1)
    neg = jnp.float32(-1e30)
    li = jnp.where(ci < EI, logits, neg)
    li = li - jnp.max(li, axis=-1, keepdims=True)
    eli = jnp.exp(li)
    pi = eli / jnp.sum(eli, axis=-1, keepdims=True)
    ls = jnp.where((ci >= EI) & (ci < EI + ES), logits, neg)
    ls = ls - jnp.max(ls, axis=-1, keepdims=True)
    els = jnp.exp(ls)
    ps = els / jnp.sum(els, axis=-1, keepdims=True)
    m1 = jnp.max(pi, axis=-1, keepdims=True)
    i1 = jnp.min(jnp.where(pi == m1, ci, 1 << 30), axis=-1, keepdims=True)
    p2 = jnp.where(ci == i1, neg, pi)
    m2 = jnp.max(p2, axis=-1, keepdims=True)
    i2 = jnp.min(jnp.where(p2 == m2, ci, 1 << 30), axis=-1, keepdims=True)
    wsum = m1 + m2
    w1n = m1 / wsum
    w2n = m2 / wsum
    wcol = jnp.where(ci == 0, w1n, 0.0) + jnp.where(ci == 1, w2n, 0.0)
    wcol = wcol + jnp.where((ci >= EI) & (ci < EI + ES), ps, 0.0)
    wcol_ref[...] = wcol
    psrep_ref[...] = jnp.concatenate(
        [jnp.broadcast_to(ps[:, EI + e:EI + e + 1], (T, 128))
         for e in range(ES)], axis=1)
    onehot = ((ci == i1) | (ci == i2)) & (ci < EI)
    cnt = jnp.sum(onehot.astype(jnp.float32), axis=0)
    tot = jnp.sum(jnp.where(ci < EI, pi, 0.0), axis=0)
    aux = EI * jnp.sum(cnt * tot) / (T * T)
    aux_ref[...] = jnp.full((8, 128), aux, jnp.float32)

    a0 = jnp.where(ci == i1, 1.0, 0.0)
    a1 = jnp.where(ci == i2, 1.0, 0.0)
    ri = lax.broadcasted_iota(jnp.int32, (SB, SB), 0)
    cj = lax.broadcasted_iota(jnp.int32, (SB, SB), 1)
    ltri = jnp.where(ri > cj, 1.0, 0.0).astype(jnp.float32)
    c0 = []
    c1 = []
    pref = jnp.zeros((1, 128), jnp.float32)
    for b in range(NB):
        blk = a0[b * SB:(b + 1) * SB, :]
        c0.append(jnp.dot(ltri, blk, preferred_element_type=jnp.float32)
                  + pref)
        pref = pref + jnp.sum(blk, axis=0, keepdims=True)
    counts0 = pref
    for b in range(NB):
        blk = a1[b * SB:(b + 1) * SB, :]
        c1.append(jnp.dot(ltri, blk, preferred_element_type=jnp.float32)
                  + pref)
        pref = pref + jnp.sum(blk, axis=0, keepdims=True)
    counts_tot = pref
    padded = jnp.ceil(counts_tot * (1.0 / BLKM)) * BLKM
    ru = lax.broadcasted_iota(jnp.int32, (128, 128), 0)
    cu = lax.broadcasted_iota(jnp.int32, (128, 128), 1)
    utri = jnp.where(ru < cu, 1.0, 0.0).astype(jnp.float32)
    offs = jnp.dot(padded, utri, preferred_element_type=jnp.float32)
    d0 = []
    d1 = []
    for b in range(NB):
        sl = slice(b * SB, (b + 1) * SB)
        t0 = offs + c0[b]
        t1 = offs + c1[b]
        d0.append(jnp.sum(jnp.where(ci[sl] == i1[sl], t0, 0.0),
                          axis=-1, keepdims=True))
        d1.append(jnp.sum(jnp.where(ci[sl] == i2[sl], t1, 0.0),
                          axis=-1, keepdims=True))
    d0c = jnp.concatenate(d0, axis=0)
    d1c = jnp.concatenate(d1, axis=0)
    dest = jnp.where(ci == 0, d0c, 0.0) + jnp.where(ci == 1, d1c, 0.0)
    dest_ref[...] = dest.astype(jnp.int32)
    bpos = (lax.broadcasted_iota(jnp.int32, (NBLK, 128), 0)
            * BLKM).astype(jnp.float32)
    cb = lax.broadcasted_iota(jnp.int32, (NBLK, 128), 1)
    hit = jnp.where((cb < EI) & (offs <= bpos), 1, 0)
    bexp = jnp.sum(hit, axis=-1, keepdims=True) - 1
    bexp_ref[...] = jnp.where(cb == 0, bexp, 0)


def _sc_dispatch(t2, dest_flat):
    mesh = plsc.VectorSubcoreMesh(core_axis_name="c", subcore_axis_name="s")

    @functools.partial(
        pl.kernel, mesh=mesh,
        out_type=jax.ShapeDtypeStruct((CAP, D), jnp.float32),
        scratch_types=[
            pltpu.VMEM((CHUNK,), jnp.int32),
            pltpu.VMEM((CHUNK, D), jnp.float32),
            pltpu.SemaphoreType.DMA,
        ],
    )
    def k(t_hbm, dest_hbm, gt_hbm, idx_v, rows_v, sem):
        wid = lax.axis_index("s") * 2 + lax.axis_index("c")
        for c in range(NCH):
            base = wid * (NCH * CHUNK) + c * CHUNK
            tokb = lax.rem(base, T)
            pltpu.sync_copy(dest_hbm.at[pl.ds(base, CHUNK)], idx_v)
            pltpu.sync_copy(t_hbm.at[pl.ds(tokb, CHUNK)], rows_v)
            pltpu.async_copy(rows_v, gt_hbm.at[idx_v], sem).wait()

    return k(t2, dest_flat)


def _sc_combine(y, dest_flat):
    mesh = plsc.VectorSubcoreMesh(core_axis_name="c", subcore_axis_name="s")

    @functools.partial(
        pl.kernel, mesh=mesh,
        out_type=jax.ShapeDtypeStruct((2 * T, D), jnp.float32),
        scratch_types=[
            pltpu.VMEM((CHUNK,), jnp.int32),
            pltpu.VMEM((CHUNK, D), jnp.float32),
            pltpu.SemaphoreType.DMA,
        ],
    )
    def k(y_hbm, dest_hbm, out_hbm, idx_v, rows_v, sem):
        wid = lax.axis_index("s") * 2 + lax.axis_index("c")
        for c in range(NCH):
            base = wid * (NCH * CHUNK) + c * CHUNK
            pltpu.sync_copy(dest_hbm.at[pl.ds(base, CHUNK)], idx_v)
            pltpu.async_copy(y_hbm.at[idx_v], rows_v, sem).wait()
            pltpu.sync_copy(rows_v, out_hbm.at[pl.ds(base, CHUNK)])

    return k(y, dest_flat)


def _gffn_body(be_ref, gt_ref, w1_ref, w2_ref, b1_ref, b2_ref, y_ref):
    h = jnp.dot(gt_ref[...].astype(jnp.bfloat16), w1_ref[0].astype(jnp.bfloat16),
                preferred_element_type=jnp.float32)
    h = jax.nn.gelu(h + b1_ref[0])
    y_ref[...] = (jnp.dot(h.astype(jnp.bfloat16),
                          w2_ref[0].astype(jnp.bfloat16),
                          preferred_element_type=jnp.float32)
                  + b2_ref[0])


def _dense_body(tb_ref, w1_ref, w2_ref, b1_ref, b2_ref, ps_ref,
                x1_ref, out_ref):
    e = pl.program_id(1)
    h = jnp.dot(tb_ref[...], w1_ref[0].astype(jnp.bfloat16),
                preferred_element_type=jnp.float32)
    h = jax.nn.gelu(h + b1_ref[0])
    y = jnp.dot(h.astype(jnp.bfloat16), w2_ref[0].astype(jnp.bfloat16),
                preferred_element_type=jnp.float32) + b2_ref[0]
    contrib = ps_ref[:, :1] * y

    @pl.when(e == 0)
    def _():
        out_ref[...] = x1_ref[...] + contrib

    @pl.when(e > 0)
    def _():
        out_ref[...] += contrib


def _combine_body(d_ref, wc_ref, y0_ref, y1_ref, out_ref):
    out_ref[...] = (d_ref[...] + wc_ref[:, 0:1] * y0_ref[...]
                    + wc_ref[:, 1:2] * y1_ref[...])


def kernel(x, ln_g, ln_b, Wq, bq, Wk, bk, Wv, bv, Wo, bo,
           gate_i, w1_i, b1_i, w2_i, b2_i, gate_s, w1_s, b1_s, w2_s, b2_s):
    f32 = jnp.float32
    bf16 = jnp.bfloat16
    lg = ln_g.reshape(1, D)
    lb = ln_b.reshape(1, D)
    x1, t, tb = pl.pallas_call(
        _attn_body,
        grid=(NOBJ,),
        in_specs=[
            pl.BlockSpec((1, N, D), lambda i: (i, 0, 0)),
            pl.BlockSpec((1, D), lambda i: (0, 0)),
            pl.BlockSpec((1, D), lambda i: (0, 0)),
            pl.BlockSpec((D, D), lambda i: (0, 0)),
            pl.BlockSpec((1, D), lambda i: (0, 0)),
            pl.BlockSpec((D, D), lambda i: (0, 0)),
            pl.BlockSpec((1, D), lambda i: (0, 0)),
            pl.BlockSpec((D, D), lambda i: (0, 0)),
            pl.BlockSpec((1, D), lambda i: (0, 0)),
            pl.BlockSpec((D, D), lambda i: (0, 0)),
            pl.BlockSpec((1, D), lambda i: (0, 0)),
        ],
        out_specs=[
            pl.BlockSpec((1, N, D), lambda i: (i, 0, 0)),
            pl.BlockSpec((1, N, D), lambda i: (i, 0, 0)),
            pl.BlockSpec((1, N, D), lambda i: (i, 0, 0)),
        ],
        out_shape=[
            jax.ShapeDtypeStruct((NOBJ, N, D), f32),
            jax.ShapeDtypeStruct((NOBJ, N, D), f32),
            jax.ShapeDtypeStruct((NOBJ, N, D), bf16),
        ],
    )(x, lg, lb, Wq, bq.reshape(1, D),
      Wk, bk.reshape(1, D), Wv, bv.reshape(1, D),
      Wo, bo.reshape(1, D))

    t2 = t.reshape(T, D)
    gatecat = jnp.zeros((D, 128), f32)
    gatecat = lax.dynamic_update_slice(gatecat, gate_i, (0, 0))
    gatecat = lax.dynamic_update_slice(gatecat, gate_s, (0, EI))
    wcol, psrep, aux_arr, destC, bexpArr = pl.pallas_call(
        _router_body,
        in_specs=[
            pl.BlockSpec((T, D), lambda: (0, 0)),
            pl.BlockSpec((D, 128), lambda: (0, 0)),
        ],
        out_specs=[
            pl.BlockSpec((T, 128), lambda: (0, 0)),
            pl.BlockSpec((T, ES * 128), lambda: (0, 0)),
            pl.BlockSpec((8, 128), lambda: (0, 0)),
            pl.BlockSpec((T, 128), lambda: (0, 0)),
            pl.BlockSpec((NBLK, 128), lambda: (0, 0)),
        ],
        out_shape=[
            jax.ShapeDtypeStruct((T, 128), f32),
            jax.ShapeDtypeStruct((T, ES * 128), f32),
            jax.ShapeDtypeStruct((8, 128), f32),
            jax.ShapeDtypeStruct((T, 128), jnp.int32),
            jax.ShapeDtypeStruct((NBLK, 128), jnp.int32),
        ],
    )(t2, gatecat)

    dest_flat = jnp.concatenate([destC[:, 0], destC[:, 1]], axis=0)
    bexp = bexpArr[:, 0]

    gt = _sc_dispatch(t2, dest_flat)

    b1s3 = b1_s.reshape(ES, 1, HID)
    b2s3 = b2_s.reshape(ES, 1, D)
    dense = pl.pallas_call(
        _dense_body,
        grid=(NM, ES),
        in_specs=[
            pl.BlockSpec((MBLK, D), lambda m, e: (m, 0)),
            pl.BlockSpec((1, D, HID), lambda m, e: (e, 0, 0)),
            pl.BlockSpec((1, HID, D), lambda m, e: (e, 0, 0)),
            pl.BlockSpec((1, 1, HID), lambda m, e: (e, 0, 0)),
            pl.BlockSpec((1, 1, D), lambda m, e: (e, 0, 0)),
            pl.BlockSpec((MBLK, 128), lambda m, e: (m, e)),
            pl.BlockSpec((MBLK, D), lambda m, e: (m, 0)),
        ],
        out_specs=pl.BlockSpec((MBLK, D), lambda m, e: (m, 0)),
        out_shape=jax.ShapeDtypeStruct((T, D), f32),
    )(tb.reshape(T, D), w1_s, w2_s, b1s3, b2s3, psrep, x1.reshape(T, D))

    b1i3 = b1_i.reshape(EI, 1, HID)
    b2i3 = b2_i.reshape(EI, 1, D)
    grid_spec = pltpu.PrefetchScalarGridSpec(
        num_scalar_prefetch=1,
        grid=(NBLK,),
        in_specs=[
            pl.BlockSpec((BLKM, D), lambda i, be: (i, 0)),
            pl.BlockSpec((1, D, HID), lambda i, be: (be[i], 0, 0)),
            pl.BlockSpec((1, HID, D), lambda i, be: (be[i], 0, 0)),
            pl.BlockSpec((1, 1, HID), lambda i, be: (be[i], 0, 0)),
            pl.BlockSpec((1, 1, D), lambda i, be: (be[i], 0, 0)),
        ],
        out_specs=pl.BlockSpec((BLKM, D), lambda i, be: (i, 0)),
    )
    y = pl.pallas_call(
        _gffn_body,
        grid_spec=grid_spec,
        out_shape=jax.ShapeDtypeStruct((CAP, D), f32),
    )(bexp, gt, w1_i, w2_i, b1i3, b2i3)

    yg = _sc_combine(y, dest_flat)

    CB = 512
    out = pl.pallas_call(
        _combine_body,
        grid=(T // CB,),
        in_specs=[
            pl.BlockSpec((CB, D), lambda m: (m, 0)),
            pl.BlockSpec((CB, 128), lambda m: (m, 0)),
            pl.BlockSpec((CB, D), lambda m: (m, 0)),
            pl.BlockSpec((CB, D), lambda m: (T // CB + m, 0)),
        ],
        out_specs=pl.BlockSpec((CB, D), lambda m: (m, 0)),
        out_shape=jax.ShapeDtypeStruct((T, D), f32),
    )(dense, wcol, yg, yg)

    return out.reshape(1, NOBJ, N, D), aux_arr[0, 0]


# speedup vs baseline: 1.0117x; 1.0117x over previous
import functools

import jax
import jax.numpy as jnp
from jax import lax
from jax.experimental import pallas as pl
from jax.experimental.pallas import tpu as pltpu
from jax.experimental.pallas import tpu_sc as plsc

D = 1024
NH = 16
HD = D // NH
EI = 16
ES = 4
HID = 1024
NOBJ = 8
N = 512
T = NOBJ * N
MBLK = 1024
NM = T // MBLK
SB = 512
NB = T // SB
BLKM = 128
CAP = 2 * T + EI * BLKM
NBLK = CAP // BLKM
NW = 32
CHUNK = 64
NCH = 2 * T // (NW * CHUNK)


def _ln(x, g, b):
    mu = jnp.mean(x, axis=-1, keepdims=True)
    xc = x - mu
    var = jnp.mean(xc * xc, axis=-1, keepdims=True)
    return xc * jax.lax.rsqrt(var + 1e-5) * g + b


def _attn_body(x_ref, lg_ref, lb_ref, wq_ref, bq_ref, wk_ref, bk_ref,
               wv_ref, bv_ref, wo_ref, bo_ref, x1_ref, t_ref, tb_ref):
    x = x_ref[0]
    g = lg_ref[...]
    b = lb_ref[...]
    xn = _ln(x, g, b).astype(jnp.bfloat16)
    q = (jnp.dot(xn, wq_ref[...].astype(jnp.bfloat16),
                 preferred_element_type=jnp.float32)
         + bq_ref[...]).astype(jnp.bfloat16)
    k = (jnp.dot(xn, wk_ref[...].astype(jnp.bfloat16),
                 preferred_element_type=jnp.float32)
         + bk_ref[...]).astype(jnp.bfloat16)
    v = (jnp.dot(xn, wv_ref[...].astype(jnp.bfloat16),
                 preferred_element_type=jnp.float32)
         + bv_ref[...]).astype(jnp.bfloat16)
    scale = 1.0 / (HD ** 0.5)
    ohs = []
    for h in range(NH):
        sl = slice(h * HD, (h + 1) * HD)
        qh = q[:, sl]
        kh = k[:, sl]
        vh = v[:, sl]
        s = lax.dot_general(qh, kh, (((1,), (1,)), ((), ())),
                            preferred_element_type=jnp.float32) * scale
        p = jnp.exp(s)
        p = p / jnp.sum(p, axis=-1, keepdims=True)
        oh = jnp.dot(p.astype(jnp.bfloat16), vh,
                     preferred_element_type=jnp.float32)
        ohs.append(oh.astype(jnp.bfloat16))
    o = jnp.concatenate(ohs, axis=1)
    xa = jnp.dot(o, wo_ref[...].astype(jnp.bfloat16),
                 preferred_element_type=jnp.float32) + bo_ref[...]
    x1 = xa + x
    x1_ref[0] = x1
    t = _ln(x1, g, b)
    t_ref[0] = t
    tb_ref[0] = t.astype(jnp.bfloat16)


def _router_body(t_ref, gate_ref, wcol_ref, psrep_ref, aux_ref, dest_ref,
                 bexp_ref):
    t = t_ref[...]
    logits = jnp.dot(t, gate_ref[...], preferred_element_type=jnp.float32)
    ci = lax.broadcasted_iota(jnp.int32, (T, 128), 1)
    neg = jnp.float32(-1e30)
    li = jnp.where(ci < EI, logits, neg)
    li = li - jnp.max(li, axis=-1, keepdims=True)
    eli = jnp.exp(li)
    pi = eli / jnp.sum(eli, axis=-1, keepdims=True)
    ls = jnp.where((ci >= EI) & (ci < EI + ES), logits, neg)
    ls = ls - jnp.max(ls, axis=-1, keepdims=True)
    els = jnp.exp(ls)
    ps = els / jnp.sum(els, axis=-1, keepdims=True)
    m1 = jnp.max(pi, axis=-1, keepdims=True)
    i1 = jnp.min(jnp.where(pi == m1, ci, 1 << 30), axis=-1, keepdims=True)
    p2 = jnp.where(ci == i1, neg, pi)
    m2 = jnp.max(p2, axis=-1, keepdims=True)
    i2 = jnp.min(jnp.where(p2 == m2, ci, 1 << 30), axis=-1, keepdims=True)
    wsum = m1 + m2
    w1n = m1 / wsum
    w2n = m2 / wsum
    wcol = jnp.where(ci == 0, w1n, 0.0) + jnp.where(ci == 1, w2n, 0.0)
    wcol = wcol + jnp.where((ci >= EI) & (ci < EI + ES), ps, 0.0)
    wcol_ref[...] = wcol
    psrep_ref[...] = jnp.concatenate(
        [jnp.broadcast_to(ps[:, EI + e:EI + e + 1], (T, 128))
         for e in range(ES)], axis=1)
    onehot = ((ci == i1) | (ci == i2)) & (ci < EI)
    cnt = jnp.sum(onehot.astype(jnp.float32), axis=0)
    tot = jnp.sum(jnp.where(ci < EI, pi, 0.0), axis=0)
    aux = EI * jnp.sum(cnt * tot) / (T * T)
    aux_ref[...] = jnp.full((8, 128), aux, jnp.float32)

    a0 = jnp.where(ci == i1, 1.0, 0.0)
    a1 = jnp.where(ci == i2, 1.0, 0.0)
    ri = lax.broadcasted_iota(jnp.int32, (SB, SB), 0)
    cj = lax.broadcasted_iota(jnp.int32, (SB, SB), 1)
    ltri = jnp.where(ri > cj, 1.0, 0.0).astype(jnp.float32)
    c0 = []
    c1 = []
    pref = jnp.zeros((1, 128), jnp.float32)
    for b in range(NB):
        blk = a0[b * SB:(b + 1) * SB, :]
        c0.append(jnp.dot(ltri, blk, preferred_element_type=jnp.float32)
                  + pref)
        pref = pref + jnp.sum(blk, axis=0, keepdims=True)
    counts0 = pref
    for b in range(NB):
        blk = a1[b * SB:(b + 1) * SB, :]
        c1.append(jnp.dot(ltri, blk, preferred_element_type=jnp.float32)
                  + pref)
        pref = pref + jnp.sum(blk, axis=0, keepdims=True)
    counts_tot = pref
    padded = jnp.ceil(counts_tot * (1.0 / BLKM)) * BLKM
    ru = lax.broadcasted_iota(jnp.int32, (128, 128), 0)
    cu = lax.broadcasted_iota(jnp.int32, (128, 128), 1)
    utri = jnp.where(ru < cu, 1.0, 0.0).astype(jnp.float32)
    offs = jnp.dot(padded, utri, preferred_element_type=jnp.float32)
    d0 = []
    d1 = []
    for b in range(NB):
        sl = slice(b * SB, (b + 1) * SB)
        t0 = offs + c0[b]
        t1 = offs + c1[b]
        d0.append(jnp.sum(jnp.where(ci[sl] == i1[sl], t0, 0.0),
                          axis=-1, keepdims=True))
        d1.append(jnp.sum(jnp.where(ci[sl] == i2[sl], t1, 0.0),
                          axis=-1, keepdims=True))
    d0c = jnp.concatenate(d0, axis=0)
    d1c = jnp.concatenate(d1, axis=0)
    dest = jnp.where(ci == 0, d0c, 0.0) + jnp.where(ci == 1, d1c, 0.0)
    dest_ref[...] = dest.astype(jnp.int32)
    bpos = (lax.broadcasted_iota(jnp.int32, (NBLK, 128), 0)
            * BLKM).astype(jnp.float32)
    cb = lax.broadcasted_iota(jnp.int32, (NBLK, 128), 1)
    hit = jnp.where((cb < EI) & (offs <= bpos), 1, 0)
    bexp = jnp.sum(hit, axis=-1, keepdims=True) - 1
    bexp_ref[...] = jnp.where(cb == 0, bexp, 0)


def _sc_dispatch(t2, dest_flat):
    mesh = plsc.VectorSubcoreMesh(core_axis_name="c", subcore_axis_name="s")

    @functools.partial(
        pl.kernel, mesh=mesh,
        out_type=jax.ShapeDtypeStruct((CAP, D), jnp.float32),
        scratch_types=[
            pltpu.VMEM((CHUNK,), jnp.int32),
            pltpu.VMEM((CHUNK, D), jnp.float32),
            pltpu.SemaphoreType.DMA,
        ],
    )
    def k(t_hbm, dest_hbm, gt_hbm, idx_v, rows_v, sem):
        wid = lax.axis_index("s") * 2 + lax.axis_index("c")
        for c in range(NCH):
            base = wid * (NCH * CHUNK) + c * CHUNK
            tokb = lax.rem(base, T)
            pltpu.sync_copy(dest_hbm.at[pl.ds(base, CHUNK)], idx_v)
            pltpu.sync_copy(t_hbm.at[pl.ds(tokb, CHUNK)], rows_v)
            pltpu.async_copy(rows_v, gt_hbm.at[idx_v], sem).wait()

    return k(t2, dest_flat)


def _sc_combine(y, dest_flat):
    mesh = plsc.VectorSubcoreMesh(core_axis_name="c", subcore_axis_name="s")

    @functools.partial(
        pl.kernel, mesh=mesh,
        out_type=jax.ShapeDtypeStruct((2 * T, D), jnp.float32),
        scratch_types=[
            pltpu.VMEM((CHUNK,), jnp.int32),
            pltpu.VMEM((CHUNK, D), jnp.float32),
            pltpu.SemaphoreType.DMA,
        ],
    )
    def k(y_hbm, dest_hbm, out_hbm, idx_v, rows_v, sem):
        wid = lax.axis_index("s") * 2 + lax.axis_index("c")
        for c in range(NCH):
            base = wid * (NCH * CHUNK) + c * CHUNK
            pltpu.sync_copy(dest_hbm.at[pl.ds(base, CHUNK)], idx_v)
            pltpu.async_copy(y_hbm.at[idx_v], rows_v, sem).wait()
            pltpu.sync_copy(rows_v, out_hbm.at[pl.ds(base, CHUNK)])

    return k(y, dest_flat)


def _gffn_body(be_ref, gt_ref, w1_ref, w2_ref, b1_ref, b2_ref, y_ref):
    h = jnp.dot(gt_ref[...].astype(jnp.bfloat16), w1_ref[0].astype(jnp.bfloat16),
                preferred_element_type=jnp.float32)
    h = jax.nn.gelu(h + b1_ref[0])
    y_ref[...] = (jnp.dot(h.astype(jnp.bfloat16),
                          w2_ref[0].astype(jnp.bfloat16),
                          preferred_element_type=jnp.float32)
                  + b2_ref[0])


def _dense_body(tb_ref, w1_ref, w2_ref, b1_ref, b2_ref, ps_ref,
                x1_ref, out_ref):
    e = pl.program_id(1)
    h = jnp.dot(tb_ref[...], w1_ref[0].astype(jnp.bfloat16),
                preferred_element_type=jnp.float32)
    h = jax.nn.gelu(h + b1_ref[0])
    y = jnp.dot(h.astype(jnp.bfloat16), w2_ref[0].astype(jnp.bfloat16),
                preferred_element_type=jnp.float32) + b2_ref[0]
    contrib = ps_ref[:, :1] * y

    @pl.when(e == 0)
    def _():
        out_ref[...] = x1_ref[...] + contrib

    @pl.when(e > 0)
    def _():
        out_ref[...] += contrib


def _combine_body(d_ref, wc_ref, y0_ref, y1_ref, out_ref):
    out_ref[...] = (d_ref[...] + wc_ref[:, 0:1] * y0_ref[...]
                    + wc_ref[:, 1:2] * y1_ref[...])


def kernel(x, ln_g, ln_b, Wq, bq, Wk, bk, Wv, bv, Wo, bo,
           gate_i, w1_i, b1_i, w2_i, b2_i, gate_s, w1_s, b1_s, w2_s, b2_s):
    f32 = jnp.float32
    bf16 = jnp.bfloat16
    lg = ln_g.reshape(1, D)
    lb = ln_b.reshape(1, D)
    x1, t, tb = pl.pallas_call(
        _attn_body,
        grid=(NOBJ,),
        in_specs=[
            pl.BlockSpec((1, N, D), lambda i: (i, 0, 0)),
            pl.BlockSpec((1, D), lambda i: (0, 0)),
            pl.BlockSpec((1, D), lambda i: (0, 0)),
            pl.BlockSpec((D, D), lambda i: (0, 0)),
            pl.BlockSpec((1, D), lambda i: (0, 0)),
            pl.BlockSpec((D, D), lambda i: (0, 0)),
            pl.BlockSpec((1, D), lambda i: (0, 0)),
            pl.BlockSpec((D, D), lambda i: (0, 0)),
            pl.BlockSpec((1, D), lambda i: (0, 0)),
            pl.BlockSpec((D, D), lambda i: (0, 0)),
            pl.BlockSpec((1, D), lambda i: (0, 0)),
        ],
        out_specs=[
            pl.BlockSpec((1, N, D), lambda i: (i, 0, 0)),
            pl.BlockSpec((1, N, D), lambda i: (i, 0, 0)),
            pl.BlockSpec((1, N, D), lambda i: (i, 0, 0)),
        ],
        out_shape=[
            jax.ShapeDtypeStruct((NOBJ, N, D), f32),
            jax.ShapeDtypeStruct((NOBJ, N, D), f32),
            jax.ShapeDtypeStruct((NOBJ, N, D), bf16),
        ],
    )(x, lg, lb, Wq, bq.reshape(1, D),
      Wk, bk.reshape(1, D), Wv, bv.reshape(1, D),
      Wo, bo.reshape(1, D))

    t2 = t.reshape(T, D)
    gatecat = jnp.zeros((D, 128), f32)
    gatecat = lax.dynamic_update_slice(gatecat, gate_i, (0, 0))
    gatecat = lax.dynamic_update_slice(gatecat, gate_s, (0, EI))
    wcol, psrep, aux_arr, destC, bexpArr = pl.pallas_call(
        _router_body,
        in_specs=[
            pl.BlockSpec((T, D), lambda: (0, 0)),
            pl.BlockSpec((D, 128), lambda: (0, 0)),
        ],
        out_specs=[
            pl.BlockSpec((T, 128), lambda: (0, 0)),
            pl.BlockSpec((T, ES * 128), lambda: (0, 0)),
            pl.BlockSpec((8, 128), lambda: (0, 0)),
            pl.BlockSpec((T, 128), lambda: (0, 0)),
            pl.BlockSpec((NBLK, 128), lambda: (0, 0)),
        ],
        out_shape=[
            jax.ShapeDtypeStruct((T, 128), f32),
            jax.ShapeDtypeStruct((T, ES * 128), f32),
            jax.ShapeDtypeStruct((8, 128), f32),
            jax.ShapeDtypeStruct((T, 128), jnp.int32),
            jax.ShapeDtypeStruct((NBLK, 128), jnp.int32),
        ],
    )(t2, gatecat)

    dest_flat = jnp.concatenate([destC[:, 0], destC[:, 1]], axis=0)
    bexp = bexpArr[:, 0]

    gt = _sc_dispatch(t2, dest_flat)

    b1s3 = b1_s.reshape(ES, 1, HID)
    b2s3 = b2_s.reshape(ES, 1, D)
    dense = pl.pallas_call(
        _dense_body,
        grid=(NM, ES),
        in_specs=[
            pl.BlockSpec((MBLK, D), lambda m, e: (m, 0)),
            pl.BlockSpec((1, D, HID), lambda m, e: (e, 0, 0)),
            pl.BlockSpec((1, HID, D), lambda m, e: (e, 0, 0)),
            pl.BlockSpec((1, 1, HID), lambda m, e: (e, 0, 0)),
            pl.BlockSpec((1, 1, D), lambda m, e: (e, 0, 0)),
            pl.BlockSpec((MBLK, 128), lambda m, e: (m, e)),
            pl.BlockSpec((MBLK, D), lambda m, e: (m, 0)),
        ],
        out_specs=pl.BlockSpec((MBLK, D), lambda m, e: (m, 0)),
        out_shape=jax.ShapeDtypeStruct((T, D), f32),
    )(tb.reshape(T, D), w1_s, w2_s, b1s3, b2s3, psrep, x1.reshape(T, D))

    b1i3 = b1_i.reshape(EI, 1, HID)
    b2i3 = b2_i.reshape(EI, 1, D)
    grid_spec = pltpu.PrefetchScalarGridSpec(
        num_scalar_prefetch=1,
        grid=(NBLK,),
        in_specs=[
            pl.BlockSpec((BLKM, D), lambda i, be: (i, 0)),
            pl.BlockSpec((1, D, HID), lambda i, be: (be[i], 0, 0)),
            pl.BlockSpec((1, HID, D), lambda i, be: (be[i], 0, 0)),
            pl.BlockSpec((1, 1, HID), lambda i, be: (be[i], 0, 0)),
            pl.BlockSpec((1, 1, D), lambda i, be: (be[i], 0, 0)),
        ],
        out_specs=pl.BlockSpec((BLKM, D), lambda i, be: (i, 0)),
    )
    y = pl.pallas_call(
        _gffn_body,
        grid_spec=grid_spec,
        out_shape=jax.ShapeDtypeStruct((CAP, D), f32),
    )(bexp, gt, w1_i, w2_i, b1i3, b2i3)

    yg = _sc_combine(y, dest_flat)

    CB = 512
    out = pl.pallas_call(
        _combine_body,
        grid=(T // CB,),
        in_specs=[
            pl.BlockSpec((CB, D), lambda m: (m, 0)),
            pl.BlockSpec((CB, 128), lambda m: (m, 0)),
            pl.BlockSpec((CB, D), lambda m: (m, 0)),
            pl.BlockSpec((CB, D), lambda m: (T // CB + m, 0)),
        ],
        out_specs=pl.BlockSpec((CB, D), lambda m: (m, 0)),
        out_shape=jax.ShapeDtypeStruct((T, D), f32),
    )(dense, wcol, yg, yg)

    return out.reshape(1, NOBJ, N, D), aux_arr[0, 0]
